# baseline (speedup 1.0000x reference)
import functools

import jax
import jax.numpy as jnp
from jax import lax
from jax.experimental import pallas as pl
from jax.experimental.pallas import tpu as pltpu
from jax.experimental.pallas import tpu_sc as plsc

L = 16


def _build_sc_call(bsz, seq, dim, bpw, nc):
    tab_words = (seq + 1) * dim
    chunk = bpw * seq
    bvec = bpw // L
    sg = 256 // dim
    ngrp = seq // sg

    mesh = plsc.VectorSubcoreMesh(core_axis_name="c", subcore_axis_name="s")

    @functools.partial(
        pl.kernel,
        mesh=mesh,
        compiler_params=pltpu.CompilerParams(needs_layout_passes=False),
        out_type=jax.ShapeDtypeStruct((seq * dim, bsz), jnp.float32),
        scratch_types=[
            pltpu.VMEM((tab_words,), jnp.float32),
            pltpu.VMEM((dim * L,), jnp.float32),
            pltpu.VMEM((chunk + L,), jnp.int32),
            pltpu.VMEM((sg * dim, bpw), jnp.float32),
            pltpu.VMEM((sg * dim, bpw), jnp.float32),
            pltpu.SemaphoreType.DMA,
            pltpu.SemaphoreType.DMA,
        ],
    )
    def sc_embed(inp_hbm, w_hbm, out_hbm, tab_v, pad_v, inp_v,
                 blk0, blk1, sem0, sem1):
        wid = lax.axis_index("s") * nc + lax.axis_index("c")
        base = wid * bpw

        pltpu.sync_copy(w_hbm.at[pl.ds(0, tab_words)], tab_v)
        pltpu.sync_copy(inp_hbm.at[pl.ds(base * seq, chunk)],
                        inp_v.at[pl.ds(0, chunk)])

        iota = lax.iota(jnp.int32, L)
        zero_v = jnp.zeros((L,), jnp.int32)

        def pad_body(d, c):
            p = plsc.load_gather(tab_v, [jnp.full((L,), d, jnp.int32)])
            pad_v[pl.ds(d * L, L)] = p
            return c
        lax.fori_loop(0, dim, pad_body, 0)

        iota_seq = iota * seq

        def do_spos(s, blk, row0):
            raw = [plsc.load_gather(inp_v, [iota_seq + (j * L * seq + s)])
                   for j in range(bvec)]
            m = [r != zero_v for r in raw]
            sbase = (s + 1) * dim

            acc = jnp.where(m[0], 0, 1)
            for j in range(1, bvec):
                acc = acc + jnp.where(m[j], 0, 1)
            npad = jnp.sum(acc)

            @pl.when(npad == 0)
            def _fast():
                def d_body(d, c):
                    t = plsc.load_gather(
                        tab_v, [jnp.full((L,), sbase + d, jnp.int32)])
                    r = row0 + d
                    for j in range(bvec):
                        blk[r, pl.ds(j * L, L)] = t
                    return c
                lax.fori_loop(0, dim, d_body, 0, unroll=4)

            @pl.when(npad != 0)
            def _slow():
                def d_body(d, c):
                    t = plsc.load_gather(
                        tab_v, [jnp.full((L,), sbase + d, jnp.int32)])
                    p = pad_v[pl.ds(d * L, L)]
                    r = row0 + d
                    for j in range(bvec):
                        blk[r, pl.ds(j * L, L)] = jnp.where(m[j], t, p)
                    return c
                lax.fori_loop(0, dim, d_body, 0, unroll=4)

        def do_group(g, blk, sem, primed):
            @pl.when(primed)
            def _wait_prev():
                pltpu.make_async_copy(
                    blk, out_hbm.at[pl.ds(0, sg * dim), pl.ds(base, bpw)],
                    sem).wait()

            for ss in range(sg):
                do_spos(g * sg + ss, blk, ss * dim)
            pltpu.async_copy(
                blk,
                out_hbm.at[pl.ds(g * sg * dim, sg * dim), pl.ds(base, bpw)],
                sem)

        def pair_body(i, c):
            do_group(2 * i, blk0, sem0, i > 0)
            do_group(2 * i + 1, blk1, sem1, i > 0)
            return c

        lax.fori_loop(0, ngrp // 2, pair_body, 0)
        pltpu.make_async_copy(
            blk0, out_hbm.at[pl.ds(0, sg * dim), pl.ds(base, bpw)],
            sem0).wait()
        pltpu.make_async_copy(
            blk1, out_hbm.at[pl.ds(0, sg * dim), pl.ds(base, bpw)],
            sem1).wait()

    return sc_embed


def kernel(input, weights):
    bsz, seq = input.shape
    dim = weights.shape[1]
    info = plsc.get_sparse_core_info()
    nc, ns = info.num_cores, info.num_subcores
    nw = nc * ns
    bpw = bsz // nw
    sc_embed = _build_sc_call(bsz, seq, dim, bpw, nc)
    out = sc_embed(input.reshape(-1), weights.reshape(-1))
    return out.reshape(seq, dim, bsz).transpose(2, 0, 1)


# speedup vs baseline: 1.0775x; 1.0775x over previous
import functools

import jax
import jax.numpy as jnp
from jax import lax
from jax.experimental import pallas as pl
from jax.experimental.pallas import tpu as pltpu
from jax.experimental.pallas import tpu_sc as plsc

L = 16


def _build_sc_call(bsz, seq, dim, bpw, nc):
    tab_words = (seq + 1) * dim
    chunk = bpw * seq
    bvec = bpw // L
    sg = 2
    ngrp = seq // sg

    mesh = plsc.VectorSubcoreMesh(core_axis_name="c", subcore_axis_name="s")

    @functools.partial(
        pl.kernel,
        mesh=mesh,
        compiler_params=pltpu.CompilerParams(needs_layout_passes=False),
        out_type=jax.ShapeDtypeStruct((seq * dim, bsz), jnp.float32),
        scratch_types=[
            pltpu.VMEM((tab_words,), jnp.float32),
            pltpu.VMEM((dim * L,), jnp.float32),
            pltpu.VMEM((chunk + L,), jnp.int32),
            pltpu.VMEM((sg * dim, bpw), jnp.float32),
            pltpu.VMEM((sg * dim, bpw), jnp.float32),
            pltpu.SemaphoreType.DMA,
            pltpu.SemaphoreType.DMA,
        ],
    )
    def sc_embed(inp_hbm, w_hbm, out_hbm, tab_v, pad_v, inp_v,
                 blk0, blk1, sem0, sem1):
        wid = lax.axis_index("s") * nc + lax.axis_index("c")
        base = wid * bpw

        pltpu.sync_copy(w_hbm.at[pl.ds(0, tab_words)], tab_v)
        pltpu.sync_copy(inp_hbm.at[pl.ds(base * seq, chunk)],
                        inp_v.at[pl.ds(0, chunk)])

        iota = lax.iota(jnp.int32, L)
        zero_v = jnp.zeros((L,), jnp.int32)

        def pad_body(d, c):
            p = plsc.load_gather(tab_v, [jnp.full((L,), d, jnp.int32)])
            pad_v[pl.ds(d * L, L)] = p
            return c
        lax.fori_loop(0, dim, pad_body, 0)

        iota_seq = iota * seq

        def do_spos(s, blk, row0):
            m = [plsc.load_gather(inp_v, [iota_seq + (j * L * seq + s)])
                 != zero_v for j in range(bvec)]
            sbase = (s + 1) * dim

            def d_body(d, c):
                t = plsc.load_gather(
                    tab_v, [jnp.full((L,), sbase + d, jnp.int32)])
                p = pad_v[pl.ds(d * L, L)]
                r = row0 + d
                for j in range(bvec):
                    blk[r, pl.ds(j * L, L)] = jnp.where(m[j], t, p)
                return c
            lax.fori_loop(0, dim, d_body, 0, unroll=4)

        def do_group(g, blk, sem, primed):
            @pl.when(primed)
            def _wait_prev():
                pltpu.make_async_copy(
                    blk, out_hbm.at[pl.ds(0, sg * dim), pl.ds(base, bpw)],
                    sem).wait()

            for ss in range(sg):
                do_spos(g * sg + ss, blk, ss * dim)
            pltpu.async_copy(
                blk,
                out_hbm.at[pl.ds(g * sg * dim, sg * dim), pl.ds(base, bpw)],
                sem)

        def pair_body(i, c):
            do_group(2 * i, blk0, sem0, i > 0)
            do_group(2 * i + 1, blk1, sem1, i > 0)
            return c

        lax.fori_loop(0, ngrp // 2, pair_body, 0)
        pltpu.make_async_copy(
            blk0, out_hbm.at[pl.ds(0, sg * dim), pl.ds(base, bpw)],
            sem0).wait()
        pltpu.make_async_copy(
            blk1, out_hbm.at[pl.ds(0, sg * dim), pl.ds(base, bpw)],
            sem1).wait()

    return sc_embed


def kernel(input, weights):
    bsz, seq = input.shape
    dim = weights.shape[1]
    info = plsc.get_sparse_core_info()
    nc, ns = info.num_cores, info.num_subcores
    nw = nc * ns
    bpw = bsz // nw
    sc_embed = _build_sc_call(bsz, seq, dim, bpw, nc)
    out = sc_embed(input.reshape(-1), weights.reshape(-1))
    return out.reshape(seq, dim, bsz).transpose(2, 0, 1)
